# baseline (speedup 1.0000x reference)
import functools

import jax
import jax.numpy as jnp
from jax import lax
from jax.experimental import pallas as pl
from jax.experimental.pallas import tpu as pltpu
from jax.experimental.pallas import tpu_sc as plsc


def _matmul_body(c_ref, w_ref, p_ref):
    p_ref[...] = jnp.dot(c_ref[...], w_ref[...],
                         preferred_element_type=jnp.float32)


def _reduce_body(p_ref, o_ref):
    x = p_ref[...]
    rows = x.shape[0]
    o_ref[...] = jnp.sum(x.reshape(rows, 8, 16), axis=2)


def _sc_gather_dot(p, c0, idx_i, idx_j, E, C, G=80):
    mesh = plsc.VectorSubcoreMesh(core_axis_name="c", subcore_axis_name="s")
    NC, NS = 2, 16
    NW = NC * NS
    per_tile = E // NW
    n_chunks = per_tile // G
    assert per_tile % G == 0 and G % 8 == 0
    n_main = (n_chunks // 2) * 2
    HB = C // 16

    scratch = [
        pltpu.VMEM((G,), jnp.int32),
        pltpu.VMEM((G,), jnp.int32),
        pltpu.VMEM((G,), jnp.int32),
        pltpu.VMEM((G,), jnp.int32),
        pltpu.VMEM((G, C), jnp.float32),
        pltpu.VMEM((G, C), jnp.float32),
        pltpu.VMEM((G, C), jnp.float32),
        pltpu.VMEM((G, C), jnp.float32),
        pltpu.VMEM((G, 16), jnp.float32),
        pltpu.VMEM((G, 16), jnp.float32),
    ] + [pltpu.SemaphoreType.DMA] * 8

    @functools.partial(
        pl.kernel,
        out_type=jax.ShapeDtypeStruct((E, 16), jnp.float32),
        scratch_types=scratch,
        mesh=mesh,
    )
    def gk(p_hbm, c_hbm, i_hbm, j_hbm, o_hbm,
           iA0, iA1, iB0, iB1, A0, A1, B0, B1, O0, O1,
           ga0, ga1, gb0, gb1, w0, w1, si0, si1):
        wid = lax.axis_index("s") * NC + lax.axis_index("c")
        base = wid * per_tile

        iA = (iA0, iA1)
        iB = (iB0, iB1)
        A = (A0, A1)
        B = (B0, B1)
        O = (O0, O1)
        gsa = (ga0, ga1)
        gsb = (gb0, gb1)
        ws = (w0, w1)
        isem = (si0, si1)

        def load_idx(k, b):
            pltpu.make_async_copy(
                i_hbm.at[pl.ds(base + k * G, G)], iA[b], isem[b]).start()
            pltpu.make_async_copy(
                j_hbm.at[pl.ds(base + k * G, G)], iB[b], isem[b]).start()

        def wait_idx(k, b):
            pltpu.make_async_copy(
                i_hbm.at[pl.ds(base + k * G, G)], iA[b], isem[b]).wait()
            pltpu.make_async_copy(
                j_hbm.at[pl.ds(base + k * G, G)], iB[b], isem[b]).wait()

        def gA(b):
            return pltpu.make_async_copy(p_hbm.at[iA[b]], A[b], gsa[b])

        def gB(b):
            return pltpu.make_async_copy(c_hbm.at[iB[b]], B[b], gsb[b])

        def wO(k, b):
            return pltpu.make_async_copy(
                O[b], o_hbm.at[pl.ds(base + k * G, G)], ws[b])

        def compute(b):
            a_ref, b_ref, o_ref = A[b], B[b], O[b]

            @pl.loop(0, G, step=4)
            def _(r):
                for u in range(4):
                    acc = (a_ref[r + u, pl.ds(0, 16)]
                           * b_ref[r + u, pl.ds(0, 16)])
                    for h in range(1, HB):
                        acc = acc + (a_ref[r + u, pl.ds(16 * h, 16)]
                                     * b_ref[r + u, pl.ds(16 * h, 16)])
                    o_ref[r + u, pl.ds(0, 16)] = acc

        for b in range(2):
            load_idx(b, b)
            wait_idx(b, b)
            gA(b).start()
            gB(b).start()

        @pl.loop(0, n_main, step=2)
        def _(j):
            for b in range(2):
                gA(b).wait()
                gB(b).wait()

                @pl.when(j + b + 2 < n_chunks)
                def _():
                    load_idx(j + b + 2, b)

                @pl.when(j + b >= 2)
                def _():
                    wO(j + b - 2, b).wait()

                compute(b)
                wO(j + b, b).start()

                @pl.when(j + b + 2 < n_chunks)
                def _():
                    wait_idx(j + b + 2, b)
                    gA(b).start()
                    gB(b).start()

        if n_chunks % 2 == 1:
            gA(0).wait()
            gB(0).wait()
            wO(n_chunks - 3, 0).wait()
            compute(0)
            wO(n_chunks - 1, 0).start()
            wO(n_chunks - 2, 1).wait()
            wO(n_chunks - 1, 0).wait()
        else:
            wO(n_chunks - 2, 0).wait()
            wO(n_chunks - 1, 1).wait()

    return gk(p, c0, idx_i, idx_j)


def kernel(c, edges, W):
    B, N, C = c.shape
    E = edges.shape[1]
    c0 = c[0]
    w0 = W[0]
    idx = edges[0].astype(jnp.int32)

    p = pl.pallas_call(
        _matmul_body,
        out_shape=jax.ShapeDtypeStruct((N, C), jnp.float32),
    )(c0, w0)

    partials = _sc_gather_dot(p, c0, idx[:, 0], idx[:, 1], E, C)

    pv = partials.reshape(E // 8, 128)
    Eb = 3200
    nblk = E // Eb
    out8 = pl.pallas_call(
        _reduce_body,
        grid=(nblk,),
        in_specs=[pl.BlockSpec((Eb // 8, 128), lambda ii: (ii, 0))],
        out_specs=pl.BlockSpec((Eb // 8, 8), lambda ii: (ii, 0)),
        out_shape=jax.ShapeDtypeStruct((E // 8, 8), jnp.float32),
    )(pv)

    return out8.reshape(1, E)


# speedup vs baseline: 1.1871x; 1.1871x over previous
import functools

import jax
import jax.numpy as jnp
from jax import lax
from jax.experimental import pallas as pl
from jax.experimental.pallas import tpu as pltpu
from jax.experimental.pallas import tpu_sc as plsc


def _matmul_body(c_ref, w_ref, p_ref):
    p_ref[...] = jnp.dot(c_ref[...], w_ref[...],
                         preferred_element_type=jnp.float32)


def _reduce_body(p_ref, o_ref):
    o_ref[...] = jnp.sum(p_ref[...], axis=1)[None, :]


def _sc_gather_dot(p, c0, idx_i, idx_j, E, C, G=80):
    mesh = plsc.VectorSubcoreMesh(core_axis_name="c", subcore_axis_name="s")
    NC, NS = 2, 16
    NW = NC * NS
    per_tile = E // NW
    n_chunks = per_tile // G
    assert per_tile % G == 0 and G % 8 == 0
    n_main = (n_chunks // 2) * 2
    HB = C // 16

    scratch = [
        pltpu.VMEM((G,), jnp.int32),
        pltpu.VMEM((G,), jnp.int32),
        pltpu.VMEM((G,), jnp.int32),
        pltpu.VMEM((G,), jnp.int32),
        pltpu.VMEM((G, C), jnp.float32),
        pltpu.VMEM((G, C), jnp.float32),
        pltpu.VMEM((G, C), jnp.float32),
        pltpu.VMEM((G, C), jnp.float32),
        pltpu.VMEM((G, 16), jnp.float32),
        pltpu.VMEM((G, 16), jnp.float32),
    ] + [pltpu.SemaphoreType.DMA] * 8

    @functools.partial(
        pl.kernel,
        out_type=jax.ShapeDtypeStruct((E, 16), jnp.float32),
        scratch_types=scratch,
        mesh=mesh,
    )
    def gk(p_hbm, c_hbm, i_hbm, j_hbm, o_hbm,
           iA0, iA1, iB0, iB1, A0, A1, B0, B1, O0, O1,
           ga0, ga1, gb0, gb1, w0, w1, si0, si1):
        wid = lax.axis_index("s") * NC + lax.axis_index("c")
        base = wid * per_tile

        iA = (iA0, iA1)
        iB = (iB0, iB1)
        A = (A0, A1)
        B = (B0, B1)
        O = (O0, O1)
        gsa = (ga0, ga1)
        gsb = (gb0, gb1)
        ws = (w0, w1)
        isem = (si0, si1)

        def load_idx(k, b):
            pltpu.make_async_copy(
                i_hbm.at[pl.ds(base + k * G, G)], iA[b], isem[b]).start()
            pltpu.make_async_copy(
                j_hbm.at[pl.ds(base + k * G, G)], iB[b], isem[b]).start()

        def wait_idx(k, b):
            pltpu.make_async_copy(
                i_hbm.at[pl.ds(base + k * G, G)], iA[b], isem[b]).wait()
            pltpu.make_async_copy(
                j_hbm.at[pl.ds(base + k * G, G)], iB[b], isem[b]).wait()

        def gA(b):
            return pltpu.make_async_copy(p_hbm.at[iA[b]], A[b], gsa[b])

        def gB(b):
            return pltpu.make_async_copy(c_hbm.at[iB[b]], B[b], gsb[b])

        def wO(k, b):
            return pltpu.make_async_copy(
                O[b], o_hbm.at[pl.ds(base + k * G, G)], ws[b])

        def compute(b):
            a_ref, b_ref, o_ref = A[b], B[b], O[b]

            @pl.loop(0, G)
            def _(r):
                acc = a_ref[r, pl.ds(0, 16)] * b_ref[r, pl.ds(0, 16)]
                for h in range(1, HB):
                    acc = acc + (a_ref[r, pl.ds(16 * h, 16)]
                                 * b_ref[r, pl.ds(16 * h, 16)])
                o_ref[r, pl.ds(0, 16)] = acc

        for b in range(2):
            load_idx(b, b)
            wait_idx(b, b)
            gA(b).start()
            gB(b).start()

        @pl.loop(0, n_main, step=2)
        def _(j):
            for b in range(2):
                gA(b).wait()
                gB(b).wait()

                @pl.when(j + b + 2 < n_chunks)
                def _():
                    load_idx(j + b + 2, b)

                @pl.when(j + b >= 2)
                def _():
                    wO(j + b - 2, b).wait()

                compute(b)
                wO(j + b, b).start()

                @pl.when(j + b + 2 < n_chunks)
                def _():
                    wait_idx(j + b + 2, b)
                    gA(b).start()
                    gB(b).start()

        if n_chunks % 2 == 1:
            gA(0).wait()
            gB(0).wait()
            wO(n_chunks - 3, 0).wait()
            compute(0)
            wO(n_chunks - 1, 0).start()
            wO(n_chunks - 2, 1).wait()
            wO(n_chunks - 1, 0).wait()
        else:
            wO(n_chunks - 2, 0).wait()
            wO(n_chunks - 1, 1).wait()

    return gk(p, c0, idx_i, idx_j)


def kernel(c, edges, W):
    B, N, C = c.shape
    E = edges.shape[1]
    c0 = c[0]
    w0 = W[0]
    idx = edges[0].astype(jnp.int32)

    p = pl.pallas_call(
        _matmul_body,
        out_shape=jax.ShapeDtypeStruct((N, C), jnp.float32),
    )(c0, w0)

    S = 2
    Es = E // S
    Eb = 3200
    nblk = Es // Eb
    parts = []
    for s in range(S):
        sl = slice(s * Es, (s + 1) * Es)
        partials = _sc_gather_dot(p, c0, idx[sl, 0], idx[sl, 1], Es, C, G=40)
        part = pl.pallas_call(
            _reduce_body,
            grid=(nblk,),
            in_specs=[pl.BlockSpec((Eb, 16), lambda ii: (ii, 0))],
            out_specs=pl.BlockSpec((1, Eb), lambda ii: (0, ii)),
            out_shape=jax.ShapeDtypeStruct((1, Es), jnp.float32),
        )(partials)
        parts.append(part)

    return jnp.concatenate(parts, axis=1)


# speedup vs baseline: 1.3374x; 1.1266x over previous
import functools

import jax
import jax.numpy as jnp
from jax import lax
from jax.experimental import pallas as pl
from jax.experimental.pallas import tpu as pltpu
from jax.experimental.pallas import tpu_sc as plsc


def _matmul_body(c_ref, w_ref, p_ref):
    p_ref[...] = jnp.dot(c_ref[...], w_ref[...],
                         preferred_element_type=jnp.float32)


def _reduce_body(p_ref, o_ref):
    o_ref[...] = jnp.sum(p_ref[...], axis=1)[None, :]


def _sc_gather_dot(p, c0, idx_i, idx_j, E, C, G=80):
    mesh = plsc.VectorSubcoreMesh(core_axis_name="c", subcore_axis_name="s")
    NC, NS = 2, 16
    NW = NC * NS
    per_tile = E // NW
    n_chunks = per_tile // G
    assert per_tile % G == 0 and G % 8 == 0
    n_main = (n_chunks // 2) * 2
    HB = C // 16

    scratch = [
        pltpu.VMEM((G,), jnp.int32),
        pltpu.VMEM((G,), jnp.int32),
        pltpu.VMEM((G,), jnp.int32),
        pltpu.VMEM((G,), jnp.int32),
        pltpu.VMEM((G, C), jnp.float32),
        pltpu.VMEM((G, C), jnp.float32),
        pltpu.VMEM((G, C), jnp.float32),
        pltpu.VMEM((G, C), jnp.float32),
        pltpu.VMEM((G, 16), jnp.float32),
        pltpu.VMEM((G, 16), jnp.float32),
    ] + [pltpu.SemaphoreType.DMA] * 8

    @functools.partial(
        pl.kernel,
        out_type=jax.ShapeDtypeStruct((E, 16), jnp.float32),
        scratch_types=scratch,
        mesh=mesh,
    )
    def gk(p_hbm, c_hbm, i_hbm, j_hbm, o_hbm,
           iA0, iA1, iB0, iB1, A0, A1, B0, B1, O0, O1,
           ga0, ga1, gb0, gb1, w0, w1, si0, si1):
        wid = lax.axis_index("s") * NC + lax.axis_index("c")
        base = wid * per_tile

        iA = (iA0, iA1)
        iB = (iB0, iB1)
        A = (A0, A1)
        B = (B0, B1)
        O = (O0, O1)
        gsa = (ga0, ga1)
        gsb = (gb0, gb1)
        ws = (w0, w1)
        isem = (si0, si1)

        def load_idx(k, b):
            pltpu.make_async_copy(
                i_hbm.at[pl.ds(base + k * G, G)], iA[b], isem[b]).start()
            pltpu.make_async_copy(
                j_hbm.at[pl.ds(base + k * G, G)], iB[b], isem[b]).start()

        def wait_idx(k, b):
            pltpu.make_async_copy(
                i_hbm.at[pl.ds(base + k * G, G)], iA[b], isem[b]).wait()
            pltpu.make_async_copy(
                j_hbm.at[pl.ds(base + k * G, G)], iB[b], isem[b]).wait()

        def gA(b):
            return pltpu.make_async_copy(p_hbm.at[iA[b]], A[b], gsa[b])

        def gB(b):
            return pltpu.make_async_copy(c_hbm.at[iB[b]], B[b], gsb[b])

        def wO(k, b):
            return pltpu.make_async_copy(
                O[b], o_hbm.at[pl.ds(base + k * G, G)], ws[b])

        def compute(b):
            a_ref, b_ref, o_ref = A[b], B[b], O[b]

            @pl.loop(0, G)
            def _(r):
                acc = a_ref[r, pl.ds(0, 16)] * b_ref[r, pl.ds(0, 16)]
                for h in range(1, HB):
                    acc = acc + (a_ref[r, pl.ds(16 * h, 16)]
                                 * b_ref[r, pl.ds(16 * h, 16)])
                o_ref[r, pl.ds(0, 16)] = acc

        for b in range(2):
            load_idx(b, b)
            wait_idx(b, b)
            gA(b).start()
            gB(b).start()

        @pl.loop(0, n_main, step=2)
        def _(j):
            for b in range(2):
                gA(b).wait()
                gB(b).wait()

                @pl.when(j + b + 2 < n_chunks)
                def _():
                    load_idx(j + b + 2, b)

                @pl.when(j + b >= 2)
                def _():
                    wO(j + b - 2, b).wait()

                compute(b)
                wO(j + b, b).start()

                @pl.when(j + b + 2 < n_chunks)
                def _():
                    wait_idx(j + b + 2, b)
                    gA(b).start()
                    gB(b).start()

        if n_chunks % 2 == 1:
            gA(0).wait()
            gB(0).wait()
            wO(n_chunks - 3, 0).wait()
            compute(0)
            wO(n_chunks - 1, 0).start()
            wO(n_chunks - 2, 1).wait()
            wO(n_chunks - 1, 0).wait()
        else:
            wO(n_chunks - 2, 0).wait()
            wO(n_chunks - 1, 1).wait()

    return gk(p, c0, idx_i, idx_j)


def kernel(c, edges, W):
    B, N, C = c.shape
    E = edges.shape[1]
    c0 = c[0]
    w0 = W[0]
    idx = edges[0].astype(jnp.int32)

    p = pl.pallas_call(
        _matmul_body,
        out_shape=jax.ShapeDtypeStruct((N, C), jnp.float32),
    )(c0, w0)

    Eb = 3200
    bounds = [0, 192000, E]
    parts = []
    for s in range(2):
        lo, hi = bounds[s], bounds[s + 1]
        Es = hi - lo
        nblk = Es // Eb
        partials = _sc_gather_dot(p, c0, idx[lo:hi, 0], idx[lo:hi, 1], Es, C)
        part = pl.pallas_call(
            _reduce_body,
            grid=(nblk,),
            in_specs=[pl.BlockSpec((Eb, 16), lambda ii: (ii, 0))],
            out_specs=pl.BlockSpec((1, Eb), lambda ii: (0, ii)),
            out_shape=jax.ShapeDtypeStruct((1, Es), jnp.float32),
        )(partials)
        parts.append(part)

    return jnp.concatenate(parts, axis=1)


# speedup vs baseline: 1.3871x; 1.0371x over previous
import functools

import jax
import jax.numpy as jnp
from jax import lax
from jax.experimental import pallas as pl
from jax.experimental.pallas import tpu as pltpu
from jax.experimental.pallas import tpu_sc as plsc


def _matmul_body(c_ref, w_ref, p_ref):
    p_ref[...] = jnp.dot(c_ref[...], w_ref[...],
                         preferred_element_type=jnp.float32)


def _reduce_body(p_ref, o_ref):
    o_ref[...] = jnp.sum(p_ref[...], axis=1)[None, :]


def _sc_gather_dot(p, c0, idx_i, idx_j, E, C, G=80):
    mesh = plsc.VectorSubcoreMesh(core_axis_name="c", subcore_axis_name="s")
    NC, NS = 2, 16
    NW = NC * NS
    per_tile = E // NW
    n_chunks = per_tile // G
    assert per_tile % G == 0 and G % 8 == 0
    n_main = (n_chunks // 2) * 2
    HB = C // 16

    scratch = [
        pltpu.VMEM((G,), jnp.int32),
        pltpu.VMEM((G,), jnp.int32),
        pltpu.VMEM((G,), jnp.int32),
        pltpu.VMEM((G,), jnp.int32),
        pltpu.VMEM((G, C), jnp.float32),
        pltpu.VMEM((G, C), jnp.float32),
        pltpu.VMEM((G, C), jnp.float32),
        pltpu.VMEM((G, C), jnp.float32),
        pltpu.VMEM((G, 16), jnp.float32),
        pltpu.VMEM((G, 16), jnp.float32),
    ] + [pltpu.SemaphoreType.DMA] * 8

    @functools.partial(
        pl.kernel,
        out_type=jax.ShapeDtypeStruct((E, 16), jnp.float32),
        scratch_types=scratch,
        mesh=mesh,
    )
    def gk(p_hbm, c_hbm, i_hbm, j_hbm, o_hbm,
           iA0, iA1, iB0, iB1, A0, A1, B0, B1, O0, O1,
           ga0, ga1, gb0, gb1, w0, w1, si0, si1):
        wid = lax.axis_index("s") * NC + lax.axis_index("c")
        base = wid * per_tile

        iA = (iA0, iA1)
        iB = (iB0, iB1)
        A = (A0, A1)
        B = (B0, B1)
        O = (O0, O1)
        gsa = (ga0, ga1)
        gsb = (gb0, gb1)
        ws = (w0, w1)
        isem = (si0, si1)

        def load_idx(k, b):
            pltpu.make_async_copy(
                i_hbm.at[pl.ds(base + k * G, G)], iA[b], isem[b]).start()
            pltpu.make_async_copy(
                j_hbm.at[pl.ds(base + k * G, G)], iB[b], isem[b]).start()

        def wait_idx(k, b):
            pltpu.make_async_copy(
                i_hbm.at[pl.ds(base + k * G, G)], iA[b], isem[b]).wait()
            pltpu.make_async_copy(
                j_hbm.at[pl.ds(base + k * G, G)], iB[b], isem[b]).wait()

        def gA(b):
            return pltpu.make_async_copy(p_hbm.at[iA[b]], A[b], gsa[b])

        def gB(b):
            return pltpu.make_async_copy(c_hbm.at[iB[b]], B[b], gsb[b])

        def wO(k, b):
            return pltpu.make_async_copy(
                O[b], o_hbm.at[pl.ds(base + k * G, G)], ws[b])

        def compute(b):
            a_ref, b_ref, o_ref = A[b], B[b], O[b]

            @pl.loop(0, G)
            def _(r):
                acc = a_ref[r, pl.ds(0, 16)] * b_ref[r, pl.ds(0, 16)]
                for h in range(1, HB):
                    acc = acc + (a_ref[r, pl.ds(16 * h, 16)]
                                 * b_ref[r, pl.ds(16 * h, 16)])
                o_ref[r, pl.ds(0, 16)] = acc

        for b in range(2):
            load_idx(b, b)
            wait_idx(b, b)
            gA(b).start()
            gB(b).start()

        @pl.loop(0, n_main, step=2)
        def _(j):
            for b in range(2):
                gA(b).wait()
                gB(b).wait()

                @pl.when(j + b + 2 < n_chunks)
                def _():
                    load_idx(j + b + 2, b)

                @pl.when(j + b >= 2)
                def _():
                    wO(j + b - 2, b).wait()

                compute(b)
                wO(j + b, b).start()

                @pl.when(j + b + 2 < n_chunks)
                def _():
                    wait_idx(j + b + 2, b)
                    gA(b).start()
                    gB(b).start()

        if n_chunks % 2 == 1:
            gA(0).wait()
            gB(0).wait()
            wO(n_chunks - 3, 0).wait()
            compute(0)
            wO(n_chunks - 1, 0).start()
            wO(n_chunks - 2, 1).wait()
            wO(n_chunks - 1, 0).wait()
        else:
            wO(n_chunks - 2, 0).wait()
            wO(n_chunks - 1, 1).wait()

    return gk(p, c0, idx_i, idx_j)


def kernel(c, edges, W):
    B, N, C = c.shape
    E = edges.shape[1]
    c0 = c[0]
    w0 = W[0]
    idx = edges[0].astype(jnp.int32)

    p = pl.pallas_call(
        _matmul_body,
        out_shape=jax.ShapeDtypeStruct((N, C), jnp.float32),
    )(c0, w0)

    Eb = 3200
    bounds = [0, 153600, 256000, E]
    parts = []
    for s in range(3):
        lo, hi = bounds[s], bounds[s + 1]
        Es = hi - lo
        nblk = Es // Eb
        partials = _sc_gather_dot(p, c0, idx[lo:hi, 0], idx[lo:hi, 1], Es, C)
        part = pl.pallas_call(
            _reduce_body,
            grid=(nblk,),
            in_specs=[pl.BlockSpec((Eb, 16), lambda ii: (ii, 0))],
            out_specs=pl.BlockSpec((1, Eb), lambda ii: (0, ii)),
            out_shape=jax.ShapeDtypeStruct((1, Es), jnp.float32),
        )(partials)
        parts.append(part)

    return jnp.concatenate(parts, axis=1)
